# baseline (speedup 1.0000x reference)
import functools

import numpy as np
import jax
import jax.numpy as jnp
from jax import lax
from jax.experimental import pallas as pl
from jax.experimental.pallas import tpu as pltpu
from jax.experimental.pallas import tpu_sc as plsc

_C = 19
_P = 8192
_IMIN = np.int32(-2147483648)


def _up_unc_body(G, seg_ref, cm_ref, up_ref, skey_ref):
    j = pl.program_id(1)
    C, h, w = seg_ref.shape[1], seg_ref.shape[2], seg_ref.shape[3]
    hs = up_ref.shape[2]
    W2 = 2 * w
    i0 = j * hs
    A = seg_ref[0, :, pl.ds(i0, hs), :]
    up_start = pl.multiple_of(jnp.maximum(i0 - 8, 0), 8)
    Wu = seg_ref[0, :, pl.ds(up_start, hs + 8), :]
    S0 = jnp.concatenate([A[:, :1, :], A[:, : hs - 1, :]], axis=1)
    Am1 = jnp.where(j == 0, S0, Wu[:, 7 : hs + 7, :])
    dn_start = pl.multiple_of(jnp.minimum(i0, h - hs - 8), 8)
    Wd = seg_ref[0, :, pl.ds(dn_start, hs + 8), :]
    S1 = jnp.concatenate([A[:, 1:, :], A[:, hs - 1 :, :]], axis=1)
    Ap1 = jnp.where(j == G - 1, S1, Wd[:, 1 : hs + 1, :])
    E = 0.25 * Am1 + 0.75 * A
    O = 0.75 * A + 0.25 * Ap1
    M = cm_ref[...]
    outs = []
    for par, V in ((0, E), (1, O)):
        V2 = V.reshape(C * hs, w)
        R = lax.dot_general(V2, M, (((1,), (0,)), ((), ())),
                            precision=lax.Precision.HIGHEST,
                            preferred_element_type=jnp.float32)
        R = R.reshape(C, hs, W2)
        up_ref[0, :, :, par, :] = R
        outs.append(R)
    for par, R in ((0, outs[0]), (1, outs[1])):
        m1 = jnp.maximum(R[0], R[1])
        m2 = jnp.minimum(R[0], R[1])
        for c in range(2, C):
            v = R[c]
            m2 = jnp.maximum(m2, jnp.minimum(m1, v))
            m1 = jnp.maximum(m1, v)
        unc = m2 - m1
        ib = lax.bitcast_convert_type(unc, jnp.int32)
        skey = jnp.where(ib >= 0, ib, jnp.bitwise_xor(~ib, jnp.int32(_IMIN)))
        skey_ref[0, :, par, :] = skey


def _col_matrix(w):
    W2 = 2 * w
    t = np.arange(W2)
    x0 = (t - 1) >> 1
    x1 = np.minimum(x0 + 1, w - 1)
    x0c = np.maximum(x0, 0)
    wx = np.where(t % 2 == 0, 0.75, 0.25).astype(np.float32)
    M = np.zeros((w, W2), np.float32)
    M[x0c, t] += 1.0 - wx
    M[x1, t] += wx
    return jnp.asarray(M)


def _upsample_unc(seg, G):
    n, C, h, w = seg.shape
    W2 = 2 * w
    hs = h // G
    M = _col_matrix(w)
    up5, skey4 = pl.pallas_call(
        functools.partial(_up_unc_body, G),
        grid=(n, G),
        in_specs=[
            pl.BlockSpec((1, C, h, w), lambda b, j: (b, 0, 0, 0)),
            pl.BlockSpec((w, W2), lambda b, j: (0, 0)),
        ],
        out_specs=[
            pl.BlockSpec((1, C, hs, 2, W2), lambda b, j: (b, 0, j, 0, 0)),
            pl.BlockSpec((1, hs, 2, W2), lambda b, j: (b, j, 0, 0)),
        ],
        out_shape=[
            jax.ShapeDtypeStruct((n, C, h, 2, W2), jnp.float32),
            jax.ShapeDtypeStruct((n, h, 2, W2), jnp.int32),
        ],
    )(seg, M)
    return up5.reshape(n, C, 2 * h, W2), skey4.reshape(n, 2 * h * W2)


def _thr_body(k, skey_ref, thr_ref):
    sk = skey_ref[0]
    imin = jnp.int32(_IMIN)

    def step(it, t_u):
        cand = t_u | (jnp.int32(1) << (31 - it))
        cand_s = cand ^ imin
        cnt = jnp.sum((sk >= cand_s).astype(jnp.int32))
        return jnp.where(cnt >= k, cand, t_u)

    t_u = lax.fori_loop(0, 32, step, jnp.int32(0))
    T = t_u ^ imin
    cnt_gt = jnp.sum((sk > T).astype(jnp.int32))
    lanes = lax.broadcasted_iota(jnp.int32, (1, 128), 1)
    thr_ref[0] = jnp.where(lanes == 0, T, jnp.where(lanes == 1, cnt_gt, 0))


def _threshold(skey, k):
    n, N = skey.shape
    sk3 = skey.reshape(n, N // 128, 128)
    thr = pl.pallas_call(
        functools.partial(_thr_body, k),
        grid=(n,),
        in_specs=[pl.BlockSpec((1, N // 128, 128), lambda b: (b, 0, 0))],
        out_specs=pl.BlockSpec((1, 1, 128), lambda b: (b, 0, 0)),
        out_shape=jax.ShapeDtypeStruct((n, 1, 128), jnp.int32),
    )(sk3)
    return thr.reshape(n, 128)


def _compact(skey, thr, k):
    n, N = skey.shape
    NW = 16
    CH = N // NW
    R = CH // 16
    PAD = k + 16
    DUMP = k
    mesh = plsc.VectorSubcoreMesh(core_axis_name="c", subcore_axis_name="s",
                                  num_cores=1)

    @functools.partial(
        pl.kernel, mesh=mesh,
        compiler_params=pltpu.CompilerParams(needs_layout_passes=False,
                                             use_tc_tiling_on_sc=False),
        out_type=jax.ShapeDtypeStruct((n * PAD,), jnp.int32),
        scratch_types=[
            pltpu.VMEM((CH,), jnp.int32),
            pltpu.VMEM((R, 16), jnp.int32),
            pltpu.VMEM((R, 16), jnp.int32),
            pltpu.VMEM((R, 16), jnp.int32),
            pltpu.VMEM((R, 16), jnp.int32),
            pltpu.VMEM((128,), jnp.int32),
            pltpu.VMEM((16, 16), jnp.int32),
            pltpu.VMEM((16,), jnp.int32),
            pltpu.VMEM((1, 16), jnp.int32),
            pltpu.VMEM_SHARED((16, 16), jnp.int32),
            pltpu.SemaphoreType.DMA,
        ],
    )
    def kern(skey_hbm, thr_hbm, out_hbm, chunk_v, gt_i, gt_c, eq_i, eq_c,
             thr_v, cnts_v, my_v, stage_v, shared, sem):
        wid = lax.axis_index("s")
        base = wid * CH
        lane = lax.iota(jnp.int32, 16)
        kb = jnp.broadcast_to(jnp.int32(k), (16,))
        dumpb = jnp.broadcast_to(jnp.int32(DUMP), (16,))
        zerob = jnp.broadcast_to(jnp.int32(0), (16,))
        for b in range(n):
            pltpu.sync_copy(skey_hbm.at[b, pl.ds(base, CH)], chunk_v)
            pltpu.sync_copy(thr_hbm.at[b], thr_v)
            T = thr_v[pl.ds(0, 16)][0]

            def pass1(i, carry):
                ngt, neq = carry
                v = chunk_v[pl.ds(i * 16, 16)]
                gidx = base + i * 16 + lane
                m_gt = v > T
                m_eq = v == T
                pg = plsc.all_reduce_population_count(m_gt)
                pe = plsc.all_reduce_population_count(m_eq)
                gt_c[i] = pg
                eq_c[i] = pe

                @pl.when(pg[0] > 0)
                def _():
                    _, sv = plsc.sort_key_val((~m_gt).astype(jnp.int32), gidx)
                    gt_i[i] = sv

                @pl.when(pe[0] > 0)
                def _():
                    _, sv = plsc.sort_key_val((~m_eq).astype(jnp.int32), gidx)
                    eq_i[i] = sv

                return ngt + pg[0], neq + pe[0]

            ngt, neq = lax.fori_loop(0, R, pass1,
                                     (jnp.int32(0), jnp.int32(0)))
            mine = (jnp.broadcast_to(ngt, (16,))
                    * (lane == 0).astype(jnp.int32)
                    + jnp.broadcast_to(neq, (16,))
                    * (lane == 1).astype(jnp.int32))
            my_v[...] = mine
            pltpu.sync_copy(my_v, shared.at[wid])
            plsc.subcore_barrier()
            pltpu.sync_copy(shared, cnts_v)
            off_gt = jnp.int32(0)
            off_eq = jnp.int32(0)
            tot_gt = jnp.int32(0)
            for wv in range(NW):
                row = cnts_v[wv]
                g = row[0]
                e = row[1]
                off_gt = off_gt + jnp.where(wv < wid, g, 0)
                off_eq = off_eq + jnp.where(wv < wid, e, 0)
                tot_gt = tot_gt + g

            def emit(ibuf, cbuf, off0):
                def row(i, off):
                    cnt = cbuf[i][0]

                    @pl.when(cnt > 0)
                    def _():
                        posv = jnp.broadcast_to(off, (16,)) + lane
                        ok = (lane < jnp.broadcast_to(cnt, (16,))) \
                            & (posv < kb)
                        pos = jnp.where(ok, posv, dumpb) + b * PAD
                        stage_v[0] = ibuf[i]
                        pltpu.async_copy(stage_v.at[0], out_hbm.at[pos],
                                         sem).wait()

                    return off + cnt

                lax.fori_loop(0, R, row, off0)

            emit(gt_i, gt_c, off_gt)
            emit(eq_i, eq_c, tot_gt + off_eq)
            plsc.subcore_barrier()

    return kern


def _gather(n, W2, hw1, w1, r1, hw2, w2, r2, k):
    NW = 32
    PPW = k // NW
    mesh = plsc.VectorSubcoreMesh(core_axis_name="c", subcore_axis_name="s")
    HW = W2 * W2
    lw2 = int(np.log2(W2))
    CP = 32

    @functools.partial(
        pl.kernel, mesh=mesh,
        compiler_params=pltpu.CompilerParams(needs_layout_passes=False,
                                             use_tc_tiling_on_sc=False),
        out_type=[
            jax.ShapeDtypeStruct((n, k, 256), jnp.float32),
            jax.ShapeDtypeStruct((n, k, 256), jnp.float32),
            jax.ShapeDtypeStruct((n, k, CP), jnp.float32),
        ],
        scratch_types=[
            pltpu.VMEM((PPW,), jnp.int32),
            pltpu.VMEM((16, 256), jnp.float32),
            pltpu.VMEM((16, 256), jnp.float32),
            pltpu.VMEM((16, 256), jnp.float32),
            pltpu.VMEM((16, 256), jnp.float32),
            pltpu.VMEM((16, 256), jnp.float32),
            pltpu.VMEM((16, CP), jnp.float32),
            pltpu.SemaphoreType.DMA,
        ],
    )
    def kern(f2_hbm, f1_hbm, seg_hbm, idx_hbm, o2_hbm, o1_hbm, oc_hbm,
             idx_v, r00, r01, r10, r11, out_v, cg_v, sem):
        core = lax.axis_index("c")
        sub = lax.axis_index("s")
        wid = sub * 2 + core
        base = wid * PPW
        lane = lax.iota(jnp.int32, 16)
        for b in range(n):
            pltpu.sync_copy(idx_hbm.at[b, pl.ds(base, PPW)], idx_v)
            for f_hbm, o_hbm, hwf, wf, r in ((f2_hbm, o2_hbm, hw2, w2, r2),
                                             (f1_hbm, o1_hbm, hw1, w1, r1)):
                lr = int(np.log2(2 * r))
                hf = hwf // wf
                inv = np.float32(1.0 / (2 * r))

                def chunk(ji, _):
                    iv = idx_v[pl.ds(ji * 16, 16)]
                    ix = iv & (W2 - 1)
                    iy = lax.shift_right_arithmetic(iv, lw2)
                    tx = 2 * ix + (1 - r)
                    x0 = lax.shift_right_arithmetic(tx, lr)
                    fx = (tx - lax.shift_left(x0, lr)).astype(jnp.float32) * inv
                    ty = 2 * iy + (1 - r)
                    y0 = lax.shift_right_arithmetic(ty, lr)
                    fy = (ty - lax.shift_left(y0, lr)).astype(jnp.float32) * inv
                    x0c = jnp.maximum(x0, 0)
                    x1c = jnp.minimum(x0 + 1, wf - 1)
                    y0c = jnp.maximum(y0, 0)
                    y1c = jnp.minimum(y0 + 1, hf - 1)
                    vx0 = (x0 >= 0).astype(jnp.float32)
                    vx1 = (x0 + 1 <= wf - 1).astype(jnp.float32)
                    vy0 = (y0 >= 0).astype(jnp.float32)
                    vy1 = (y0 + 1 <= hf - 1).astype(jnp.float32)
                    wx0 = (1.0 - fx) * vx0
                    wx1 = fx * vx1
                    wy0 = (1.0 - fy) * vy0
                    wy1 = fy * vy1
                    w00 = wx0 * wy0
                    w01 = wx1 * wy0
                    w10 = wx0 * wy1
                    w11 = wx1 * wy1
                    rb = b * hwf + y0c * wf
                    rt = b * hwf + y1c * wf
                    d00 = pltpu.async_copy(f_hbm.at[rb + x0c], r00, sem)
                    d01 = pltpu.async_copy(f_hbm.at[rb + x1c], r01, sem)
                    d10 = pltpu.async_copy(f_hbm.at[rt + x0c], r10, sem)
                    d11 = pltpu.async_copy(f_hbm.at[rt + x1c], r11, sem)
                    d00.wait()
                    d01.wait()
                    d10.wait()
                    d11.wait()

                    def comb(cc, _):
                        s = cc * 16
                        for p in range(16):
                            acc = (r00[p, pl.ds(s, 16)] * w00[p]
                                   + r01[p, pl.ds(s, 16)] * w01[p]
                                   + r10[p, pl.ds(s, 16)] * w10[p]
                                   + r11[p, pl.ds(s, 16)] * w11[p])
                            out_v[p, pl.ds(s, 16)] = acc
                        return 0

                    lax.fori_loop(0, 16, comb, 0)
                    pltpu.sync_copy(out_v, o_hbm.at[b, pl.ds(base + ji * 16, 16)])
                    return 0

                lax.fori_loop(0, PPW // 16, chunk, 0)
            def cchunk(ji, _):
                iv = idx_v[pl.ds(ji * 16, 16)] + b * HW
                pltpu.async_copy(seg_hbm.at[iv], cg_v, sem).wait()
                pltpu.sync_copy(cg_v, oc_hbm.at[b, pl.ds(base + ji * 16, 16)])
                return 0

            lax.fori_loop(0, PPW // 16, cchunk, 0)

    return kern


def _scatter(n, HW, k):
    NW = 16
    PPW = k // NW
    TOT = n * _C * HW
    span = TOT // NW
    CS = HW // 8
    NQ = PPW // 128
    PR = k // 128
    mesh = plsc.VectorSubcoreMesh(core_axis_name="c", subcore_axis_name="s",
                                  num_cores=1)

    @functools.partial(
        pl.kernel, mesh=mesh,
        compiler_params=pltpu.CompilerParams(needs_layout_passes=False,
                                             use_tc_tiling_on_sc=False),
        out_type=jax.ShapeDtypeStruct((TOT,), jnp.float32),
        scratch_types=[
            pltpu.VMEM((CS,), jnp.float32),
            pltpu.VMEM((PPW,), jnp.int32),
            pltpu.VMEM((_C * NQ, 128), jnp.int32),
            pltpu.VMEM((_C * NQ, 128), jnp.float32),
            pltpu.SemaphoreType.DMA,
        ],
    )
    def kern(seg_hbm, pl2_hbm, idx_hbm, out_hbm, buf_v, idx_v, ci_v, val_v,
             sem):
        wid = lax.axis_index("s")

        def cp(i, _):
            start = wid * span + i * CS
            pltpu.sync_copy(seg_hbm.at[pl.ds(start, CS)], buf_v)
            pltpu.sync_copy(buf_v, out_hbm.at[pl.ds(start, CS)])
            return 0

        lax.fori_loop(0, span // CS, cp, 0)
        plsc.subcore_barrier()
        base = wid * PPW
        for b in range(n):
            pltpu.sync_copy(idx_hbm.at[b, pl.ds(base, PPW)], idx_v)
            for c in range(_C):
                row0 = (b * _C + c) * PR + base // 128
                pltpu.sync_copy(pl2_hbm.at[pl.ds(row0, NQ)],
                                val_v.at[pl.ds(c * NQ, NQ)])
                off = (b * _C + c) * HW
                for q in range(NQ):

                    def ci_fill(t, _):
                        ci_v[NQ * c + q, pl.ds(t * 16, 16)] = (
                            idx_v[pl.ds(q * 128 + t * 16, 16)] + off)
                        return 0

                    lax.fori_loop(0, 8, ci_fill, 0)
                dmas = []
                for q in range(NQ):
                    dmas.append(pltpu.async_copy(
                        val_v.at[NQ * c + q],
                        out_hbm.at[ci_v.at[NQ * c + q]], sem))
                for d in dmas:
                    d.wait()

    return kern


def _mlp_body(f2_ref, f1_ref, cp_ref, w1a_ref, w1b_ref, w1c_ref, b1_ref,
              w2h_ref, w2c_ref, b2_ref, w3h_ref, w3c_ref, b3_ref,
              wph_ref, wpc_ref, bp_ref, out_ref):
    x2 = f2_ref[0]
    x1 = f1_ref[0]
    cP = cp_ref[0]
    dnT = (((1,), (1,)), ((), ()))

    def dot(a, b, dn):
        return lax.dot_general(a, b, dn, preferred_element_type=jnp.float32)

    h = dot(x2, w1a_ref[...], dnT) + dot(x1, w1b_ref[...], dnT) \
        + dot(cP, w1c_ref[...], dnT) + b1_ref[...]
    h = jnp.maximum(h, 0.0)
    h = dot(h, w2h_ref[...], dnT) + dot(cP, w2c_ref[...], dnT) + b2_ref[...]
    h = jnp.maximum(h, 0.0)
    h = dot(h, w3h_ref[...], dnT) + dot(cP, w3c_ref[...], dnT) + b3_ref[...]
    h = jnp.maximum(h, 0.0)
    outT = dot(wph_ref[...], h, dnT) \
        + dot(wpc_ref[...], cP, dnT) \
        + bp_ref[...][:, :1]
    out_ref[0] = outT


def _mlp(f2, f1, cP, params, k):
    n = f2.shape[0]
    BLK = 512
    (w1a, w1b, w1c, b1, w2h, w2c, b2, w3h, w3c, b3, wph, wpc, bp) = params
    wspec = lambda shp: pl.BlockSpec(shp, lambda b, j: tuple(0 for _ in shp))
    plT = pl.pallas_call(
        _mlp_body,
        grid=(n, k // BLK),
        in_specs=[
            pl.BlockSpec((1, BLK, 256), lambda b, j: (b, j, 0)),
            pl.BlockSpec((1, BLK, 256), lambda b, j: (b, j, 0)),
            pl.BlockSpec((1, BLK, 32), lambda b, j: (b, j, 0)),
            wspec((256, 256)), wspec((256, 256)), wspec((256, 32)),
            wspec((1, 256)),
            wspec((256, 256)), wspec((256, 32)), wspec((1, 256)),
            wspec((256, 256)), wspec((256, 32)), wspec((1, 256)),
            wspec((_C, 256)), wspec((_C, 32)), wspec((_C, 128)),
        ],
        out_specs=pl.BlockSpec((1, _C, BLK), lambda b, j: (b, 0, j)),
        out_shape=jax.ShapeDtypeStruct((n, _C, k), jnp.float32),
    )(f2, f1, cP, w1a, w1b, w1c, b1, w2h, w2c, b2, w3h, w3c, b3, wph, wpc, bp)
    return plT


def kernel(feat1, feat2, coarse_logits, fc1_w, fc1_b, fc2_w, fc2_b,
           fc3_w, fc3_b, pred_w, pred_b):
    n = feat1.shape[0]
    h1 = feat1.shape[2]
    h2f = feat2.shape[2]
    hw1, hw2 = h1 * h1, h2f * h2f
    f1T = feat1.reshape(n, 256, hw1).transpose(0, 2, 1).reshape(n * hw1, 256)
    f2T = feat2.reshape(n, 256, hw2).transpose(0, 2, 1).reshape(n * hw2, 256)
    def pad32(w):
        return jnp.pad(w, ((0, 0), (0, 32 - _C)))

    params = (
        fc1_w[:, :256], fc1_w[:, 256:512], pad32(fc1_w[:, 512:]),
        fc1_b.reshape(1, 256),
        fc2_w[:, :256], pad32(fc2_w[:, 256:]), fc2_b.reshape(1, 256),
        fc3_w[:, :256], pad32(fc3_w[:, 256:]), fc3_b.reshape(1, 256),
        pred_w[:, :256], pad32(pred_w[:, 256:]),
        jnp.broadcast_to(pred_b[:, None], (_C, 128)),
    )
    seg = coarse_logits
    for G in (2, 8):
        hprev = seg.shape[2]
        W2 = 2 * hprev
        HW = W2 * W2
        seg_up, skey = _upsample_unc(seg, G)
        thr = _threshold(skey, _P)
        idxf = _compact(skey, thr, _P)(skey, thr)
        idx = idxf.reshape(n, _P + 16)[:, :_P]
        seg_pm = jnp.pad(seg_up.transpose(0, 2, 3, 1),
                         ((0, 0), (0, 0), (0, 0), (0, 32 - _C))
                         ).reshape(n * HW, 32)
        f2g, f1g, cPg = _gather(n, W2, hw1, h1, W2 // h1, hw2, h2f, W2 // h2f,
                                _P)(f2T, f1T, seg_pm, idx)
        plT = _mlp(f2g, f1g, cPg, params, _P)
        out_flat = _scatter(n, HW, _P)(seg_up.reshape(n * _C * HW),
                                       plT.reshape(n * _C * _P // 128, 128),
                                       idx)
        seg = out_flat.reshape(n, _C, W2, W2)
    return seg


# speedup vs baseline: 1.8920x; 1.8920x over previous
import functools

import numpy as np
import jax
import jax.numpy as jnp
from jax import lax
from jax.experimental import pallas as pl
from jax.experimental.pallas import tpu as pltpu
from jax.experimental.pallas import tpu_sc as plsc

_C = 19
_P = 8192
_IMIN = np.int32(-2147483648)


def _up_unc_body(G, seg_ref, cm_ref, up_ref, skey_ref):
    j = pl.program_id(1)
    C, h, w = seg_ref.shape[1], seg_ref.shape[2], seg_ref.shape[3]
    hs = up_ref.shape[2]
    W2 = 2 * w
    i0 = j * hs
    A = seg_ref[0, :, pl.ds(i0, hs), :]
    up_start = pl.multiple_of(jnp.maximum(i0 - 8, 0), 8)
    Wu = seg_ref[0, :, pl.ds(up_start, hs + 8), :]
    S0 = jnp.concatenate([A[:, :1, :], A[:, : hs - 1, :]], axis=1)
    Am1 = jnp.where(j == 0, S0, Wu[:, 7 : hs + 7, :])
    dn_start = pl.multiple_of(jnp.minimum(i0, h - hs - 8), 8)
    Wd = seg_ref[0, :, pl.ds(dn_start, hs + 8), :]
    S1 = jnp.concatenate([A[:, 1:, :], A[:, hs - 1 :, :]], axis=1)
    Ap1 = jnp.where(j == G - 1, S1, Wd[:, 1 : hs + 1, :])
    E = 0.25 * Am1 + 0.75 * A
    O = 0.75 * A + 0.25 * Ap1
    M = cm_ref[...]
    outs = []
    for par, V in ((0, E), (1, O)):
        V2 = V.reshape(C * hs, w)
        R = lax.dot_general(V2, M, (((1,), (0,)), ((), ())),
                            precision=lax.Precision.HIGHEST,
                            preferred_element_type=jnp.float32)
        R = R.reshape(C, hs, W2)
        up_ref[0, :, :, par, :] = R
        outs.append(R)
    for par, R in ((0, outs[0]), (1, outs[1])):
        m1 = jnp.maximum(R[0], R[1])
        m2 = jnp.minimum(R[0], R[1])
        for c in range(2, C):
            v = R[c]
            m2 = jnp.maximum(m2, jnp.minimum(m1, v))
            m1 = jnp.maximum(m1, v)
        unc = m2 - m1
        ib = lax.bitcast_convert_type(unc, jnp.int32)
        skey = jnp.where(ib >= 0, ib, jnp.bitwise_xor(~ib, jnp.int32(_IMIN)))
        skey_ref[0, :, par, :] = skey


def _col_matrix(w):
    W2 = 2 * w
    t = np.arange(W2)
    x0 = (t - 1) >> 1
    x1 = np.minimum(x0 + 1, w - 1)
    x0c = np.maximum(x0, 0)
    wx = np.where(t % 2 == 0, 0.75, 0.25).astype(np.float32)
    M = np.zeros((w, W2), np.float32)
    M[x0c, t] += 1.0 - wx
    M[x1, t] += wx
    return jnp.asarray(M)


def _upsample_unc(seg, G):
    n, C, h, w = seg.shape
    W2 = 2 * w
    hs = h // G
    M = _col_matrix(w)
    up5, skey4 = pl.pallas_call(
        functools.partial(_up_unc_body, G),
        grid=(n, G),
        in_specs=[
            pl.BlockSpec((1, C, h, w), lambda b, j: (b, 0, 0, 0)),
            pl.BlockSpec((w, W2), lambda b, j: (0, 0)),
        ],
        out_specs=[
            pl.BlockSpec((1, C, hs, 2, W2), lambda b, j: (b, 0, j, 0, 0)),
            pl.BlockSpec((1, hs, 2, W2), lambda b, j: (b, j, 0, 0)),
        ],
        out_shape=[
            jax.ShapeDtypeStruct((n, C, h, 2, W2), jnp.float32),
            jax.ShapeDtypeStruct((n, h, 2, W2), jnp.int32),
        ],
    )(seg, M)
    return up5.reshape(n, C, 2 * h, W2), skey4.reshape(n, 2 * h * W2)


def _thr_body(k, skey_ref, thr_ref):
    sk = skey_ref[0]
    imin = jnp.int32(_IMIN)

    def step(it, t_u):
        cand = t_u | (jnp.int32(1) << (31 - it))
        cand_s = cand ^ imin
        cnt = jnp.sum((sk >= cand_s).astype(jnp.int32))
        return jnp.where(cnt >= k, cand, t_u)

    t_u = lax.fori_loop(0, 32, step, jnp.int32(0))
    T = t_u ^ imin
    cnt_gt = jnp.sum((sk > T).astype(jnp.int32))
    lanes = lax.broadcasted_iota(jnp.int32, (1, 128), 1)
    thr_ref[0] = jnp.where(lanes == 0, T, jnp.where(lanes == 1, cnt_gt, 0))


def _threshold(skey, k):
    n, N = skey.shape
    sk3 = skey.reshape(n, N // 128, 128)
    thr = pl.pallas_call(
        functools.partial(_thr_body, k),
        grid=(n,),
        in_specs=[pl.BlockSpec((1, N // 128, 128), lambda b: (b, 0, 0))],
        out_specs=pl.BlockSpec((1, 1, 128), lambda b: (b, 0, 0)),
        out_shape=jax.ShapeDtypeStruct((n, 1, 128), jnp.int32),
    )(sk3)
    return thr.reshape(n, 128)


def _compact(skey, thr, k):
    n, N = skey.shape
    NW = 16
    CH = N // NW
    R = CH // 16
    PAD = k + 16 * NW
    mesh = plsc.VectorSubcoreMesh(core_axis_name="c", subcore_axis_name="s",
                                  num_cores=1)

    @functools.partial(
        pl.kernel, mesh=mesh,
        compiler_params=pltpu.CompilerParams(needs_layout_passes=False,
                                             use_tc_tiling_on_sc=False),
        out_type=jax.ShapeDtypeStruct((n * PAD,), jnp.int32),
        scratch_types=[
            pltpu.VMEM((CH,), jnp.int32),
            pltpu.VMEM((R, 16), jnp.int32),
            pltpu.VMEM((R, 16), jnp.int32),
            pltpu.VMEM((R, 16), jnp.int32),
            pltpu.VMEM((R, 16), jnp.int32),
            pltpu.VMEM((128,), jnp.int32),
            pltpu.VMEM((16, 16), jnp.int32),
            pltpu.VMEM((16,), jnp.int32),
            pltpu.VMEM((1, 16), jnp.int32),
            pltpu.VMEM_SHARED((16, 16), jnp.int32),
            pltpu.SemaphoreType.DMA,
        ],
    )
    def kern(skey_hbm, thr_hbm, out_hbm, chunk_v, gt_i, gt_c, eq_i, eq_c,
             thr_v, cnts_v, my_v, stage_v, shared, sem):
        wid = lax.axis_index("s")
        base = wid * CH
        lane = lax.iota(jnp.int32, 16)
        kb = jnp.broadcast_to(jnp.int32(k), (16,))
        dumpb = k + wid * 16 + lane
        for b in range(n):
            pltpu.sync_copy(skey_hbm.at[b, pl.ds(base, CH)], chunk_v)
            pltpu.sync_copy(thr_hbm.at[b], thr_v)
            T = thr_v[pl.ds(0, 16)][0]

            def pass1(i, carry):
                ngt, neq = carry
                v = chunk_v[pl.ds(i * 16, 16)]
                gidx = base + i * 16 + lane
                m_gt = v > T
                m_eq = v == T
                pg = plsc.all_reduce_population_count(m_gt)
                pe = plsc.all_reduce_population_count(m_eq)
                gt_c[i] = pg
                eq_c[i] = pe

                @pl.when(pg[0] > 0)
                def _():
                    _, sv = plsc.sort_key_val((~m_gt).astype(jnp.int32), gidx)
                    gt_i[i] = sv

                @pl.when(pe[0] > 0)
                def _():
                    _, sv = plsc.sort_key_val((~m_eq).astype(jnp.int32), gidx)
                    eq_i[i] = sv

                return ngt + pg[0], neq + pe[0]

            ngt, neq = lax.fori_loop(0, R, pass1,
                                     (jnp.int32(0), jnp.int32(0)))
            mine = (jnp.broadcast_to(ngt, (16,))
                    * (lane == 0).astype(jnp.int32)
                    + jnp.broadcast_to(neq, (16,))
                    * (lane == 1).astype(jnp.int32))
            my_v[...] = mine
            pltpu.sync_copy(my_v, shared.at[wid])
            plsc.subcore_barrier()
            pltpu.sync_copy(shared, cnts_v)
            off_gt = jnp.int32(0)
            off_eq = jnp.int32(0)
            tot_gt = jnp.int32(0)
            for wv in range(NW):
                row = cnts_v[wv]
                g = row[0]
                e = row[1]
                off_gt = off_gt + jnp.where(wv < wid, g, 0)
                off_eq = off_eq + jnp.where(wv < wid, e, 0)
                tot_gt = tot_gt + g

            def emit(ibuf, cbuf, off0):
                def row(i, off):
                    cnt = cbuf[i][0]

                    @pl.when(cnt > 0)
                    def _():
                        posv = jnp.broadcast_to(off, (16,)) + lane
                        ok = (lane < jnp.broadcast_to(cnt, (16,))) \
                            & (posv < kb)
                        pos = jnp.where(ok, posv, dumpb) + b * PAD
                        stage_v[0] = ibuf[i]
                        pltpu.async_copy(stage_v.at[0], out_hbm.at[pos],
                                         sem).wait()

                    return off + cnt

                lax.fori_loop(0, R, row, off0)

            emit(gt_i, gt_c, off_gt)
            emit(eq_i, eq_c, tot_gt + off_eq)
            plsc.subcore_barrier()

    return kern


def _gather(n, W2, hw1, w1, r1, hw2, w2, r2, k):
    NW = 32
    PPW = k // NW
    mesh = plsc.VectorSubcoreMesh(core_axis_name="c", subcore_axis_name="s")
    HW = W2 * W2
    lw2 = int(np.log2(W2))
    CP = 32

    @functools.partial(
        pl.kernel, mesh=mesh,
        compiler_params=pltpu.CompilerParams(needs_layout_passes=False,
                                             use_tc_tiling_on_sc=False),
        out_type=[
            jax.ShapeDtypeStruct((n, k, 256), jnp.float32),
            jax.ShapeDtypeStruct((n, k, 256), jnp.float32),
            jax.ShapeDtypeStruct((n, k, CP), jnp.float32),
        ],
        scratch_types=[
            pltpu.VMEM((PPW,), jnp.int32),
            pltpu.VMEM((16, 256), jnp.float32),
            pltpu.VMEM((16, 256), jnp.float32),
            pltpu.VMEM((16, 256), jnp.float32),
            pltpu.VMEM((16, 256), jnp.float32),
            pltpu.VMEM((16, 256), jnp.float32),
            pltpu.VMEM((16, CP), jnp.float32),
            pltpu.SemaphoreType.DMA,
        ],
    )
    def kern(f2_hbm, f1_hbm, seg_hbm, idx_hbm, o2_hbm, o1_hbm, oc_hbm,
             idx_v, r00, r01, r10, r11, out_v, cg_v, sem):
        core = lax.axis_index("c")
        sub = lax.axis_index("s")
        wid = sub * 2 + core
        base = wid * PPW
        lane = lax.iota(jnp.int32, 16)
        for b in range(n):
            pltpu.sync_copy(idx_hbm.at[b, pl.ds(base, PPW)], idx_v)
            for f_hbm, o_hbm, hwf, wf, r in ((f2_hbm, o2_hbm, hw2, w2, r2),
                                             (f1_hbm, o1_hbm, hw1, w1, r1)):
                lr = int(np.log2(2 * r))
                hf = hwf // wf
                inv = np.float32(1.0 / (2 * r))

                def chunk(ji, _):
                    iv = idx_v[pl.ds(ji * 16, 16)]
                    ix = iv & (W2 - 1)
                    iy = lax.shift_right_arithmetic(iv, lw2)
                    tx = 2 * ix + (1 - r)
                    x0 = lax.shift_right_arithmetic(tx, lr)
                    fx = (tx - lax.shift_left(x0, lr)).astype(jnp.float32) * inv
                    ty = 2 * iy + (1 - r)
                    y0 = lax.shift_right_arithmetic(ty, lr)
                    fy = (ty - lax.shift_left(y0, lr)).astype(jnp.float32) * inv
                    x0c = jnp.maximum(x0, 0)
                    x1c = jnp.minimum(x0 + 1, wf - 1)
                    y0c = jnp.maximum(y0, 0)
                    y1c = jnp.minimum(y0 + 1, hf - 1)
                    vx0 = (x0 >= 0).astype(jnp.float32)
                    vx1 = (x0 + 1 <= wf - 1).astype(jnp.float32)
                    vy0 = (y0 >= 0).astype(jnp.float32)
                    vy1 = (y0 + 1 <= hf - 1).astype(jnp.float32)
                    wx0 = (1.0 - fx) * vx0
                    wx1 = fx * vx1
                    wy0 = (1.0 - fy) * vy0
                    wy1 = fy * vy1
                    w00 = wx0 * wy0
                    w01 = wx1 * wy0
                    w10 = wx0 * wy1
                    w11 = wx1 * wy1
                    rb = b * hwf + y0c * wf
                    rt = b * hwf + y1c * wf
                    d00 = pltpu.async_copy(f_hbm.at[rb + x0c], r00, sem)
                    d01 = pltpu.async_copy(f_hbm.at[rb + x1c], r01, sem)
                    d10 = pltpu.async_copy(f_hbm.at[rt + x0c], r10, sem)
                    d11 = pltpu.async_copy(f_hbm.at[rt + x1c], r11, sem)
                    d00.wait()
                    d01.wait()
                    d10.wait()
                    d11.wait()

                    def comb(cc, _):
                        s = cc * 16
                        for p in range(16):
                            acc = (r00[p, pl.ds(s, 16)] * w00[p]
                                   + r01[p, pl.ds(s, 16)] * w01[p]
                                   + r10[p, pl.ds(s, 16)] * w10[p]
                                   + r11[p, pl.ds(s, 16)] * w11[p])
                            out_v[p, pl.ds(s, 16)] = acc
                        return 0

                    lax.fori_loop(0, 16, comb, 0)
                    pltpu.sync_copy(out_v, o_hbm.at[b, pl.ds(base + ji * 16, 16)])
                    return 0

                lax.fori_loop(0, PPW // 16, chunk, 0)
            def cchunk(ji, _):
                iv = idx_v[pl.ds(ji * 16, 16)] + b * HW
                pltpu.async_copy(seg_hbm.at[iv], cg_v, sem).wait()
                pltpu.sync_copy(cg_v, oc_hbm.at[b, pl.ds(base + ji * 16, 16)])
                return 0

            lax.fori_loop(0, PPW // 16, cchunk, 0)

    return kern


def _scatter(n, HW, k):
    NW = 16
    PPW = k // NW
    TOT = n * _C * HW
    span = TOT // NW
    CS = HW // 8
    NQ = PPW // 128
    PR = k // 128
    mesh = plsc.VectorSubcoreMesh(core_axis_name="c", subcore_axis_name="s",
                                  num_cores=1)

    @functools.partial(
        pl.kernel, mesh=mesh,
        compiler_params=pltpu.CompilerParams(needs_layout_passes=False,
                                             use_tc_tiling_on_sc=False),
        out_type=jax.ShapeDtypeStruct((TOT,), jnp.float32),
        scratch_types=[
            pltpu.VMEM((CS,), jnp.float32),
            pltpu.VMEM((PPW,), jnp.int32),
            pltpu.VMEM((_C * NQ, 128), jnp.int32),
            pltpu.VMEM((_C * NQ, 128), jnp.float32),
            pltpu.SemaphoreType.DMA,
        ],
    )
    def kern(seg_hbm, pl2_hbm, idx_hbm, out_hbm, buf_v, idx_v, ci_v, val_v,
             sem):
        wid = lax.axis_index("s")

        def cp(i, _):
            start = wid * span + i * CS
            pltpu.sync_copy(seg_hbm.at[pl.ds(start, CS)], buf_v)
            pltpu.sync_copy(buf_v, out_hbm.at[pl.ds(start, CS)])
            return 0

        lax.fori_loop(0, span // CS, cp, 0)
        plsc.subcore_barrier()
        base = wid * PPW
        for b in range(n):
            pltpu.sync_copy(idx_hbm.at[b, pl.ds(base, PPW)], idx_v)
            for c in range(_C):
                row0 = (b * _C + c) * PR + base // 128
                pltpu.sync_copy(pl2_hbm.at[pl.ds(row0, NQ)],
                                val_v.at[pl.ds(c * NQ, NQ)])
                off = (b * _C + c) * HW
                for q in range(NQ):

                    def ci_fill(t, _):
                        ci_v[NQ * c + q, pl.ds(t * 16, 16)] = (
                            idx_v[pl.ds(q * 128 + t * 16, 16)] + off)
                        return 0

                    lax.fori_loop(0, 8, ci_fill, 0)
                dmas = []
                for q in range(NQ):
                    dmas.append(pltpu.async_copy(
                        val_v.at[NQ * c + q],
                        out_hbm.at[ci_v.at[NQ * c + q]], sem))
                for d in dmas:
                    d.wait()

    return kern


def _mlp_body(f2_ref, f1_ref, cp_ref, w1a_ref, w1b_ref, w1c_ref, b1_ref,
              w2h_ref, w2c_ref, b2_ref, w3h_ref, w3c_ref, b3_ref,
              wph_ref, wpc_ref, bp_ref, out_ref):
    x2 = f2_ref[0]
    x1 = f1_ref[0]
    cP = cp_ref[0]
    dnT = (((1,), (1,)), ((), ()))

    def dot(a, b, dn):
        return lax.dot_general(a, b, dn, preferred_element_type=jnp.float32)

    h = dot(x2, w1a_ref[...], dnT) + dot(x1, w1b_ref[...], dnT) \
        + dot(cP, w1c_ref[...], dnT) + b1_ref[...]
    h = jnp.maximum(h, 0.0)
    h = dot(h, w2h_ref[...], dnT) + dot(cP, w2c_ref[...], dnT) + b2_ref[...]
    h = jnp.maximum(h, 0.0)
    h = dot(h, w3h_ref[...], dnT) + dot(cP, w3c_ref[...], dnT) + b3_ref[...]
    h = jnp.maximum(h, 0.0)
    outT = dot(wph_ref[...], h, dnT) \
        + dot(wpc_ref[...], cP, dnT) \
        + bp_ref[...][:, :1]
    out_ref[0] = outT


def _mlp(f2, f1, cP, params, k):
    n = f2.shape[0]
    BLK = 512
    (w1a, w1b, w1c, b1, w2h, w2c, b2, w3h, w3c, b3, wph, wpc, bp) = params
    wspec = lambda shp: pl.BlockSpec(shp, lambda b, j: tuple(0 for _ in shp))
    plT = pl.pallas_call(
        _mlp_body,
        grid=(n, k // BLK),
        in_specs=[
            pl.BlockSpec((1, BLK, 256), lambda b, j: (b, j, 0)),
            pl.BlockSpec((1, BLK, 256), lambda b, j: (b, j, 0)),
            pl.BlockSpec((1, BLK, 32), lambda b, j: (b, j, 0)),
            wspec((256, 256)), wspec((256, 256)), wspec((256, 32)),
            wspec((1, 256)),
            wspec((256, 256)), wspec((256, 32)), wspec((1, 256)),
            wspec((256, 256)), wspec((256, 32)), wspec((1, 256)),
            wspec((_C, 256)), wspec((_C, 32)), wspec((_C, 128)),
        ],
        out_specs=pl.BlockSpec((1, _C, BLK), lambda b, j: (b, 0, j)),
        out_shape=jax.ShapeDtypeStruct((n, _C, k), jnp.float32),
    )(f2, f1, cP, w1a, w1b, w1c, b1, w2h, w2c, b2, w3h, w3c, b3, wph, wpc, bp)
    return plT


def kernel(feat1, feat2, coarse_logits, fc1_w, fc1_b, fc2_w, fc2_b,
           fc3_w, fc3_b, pred_w, pred_b):
    n = feat1.shape[0]
    h1 = feat1.shape[2]
    h2f = feat2.shape[2]
    hw1, hw2 = h1 * h1, h2f * h2f
    f1T = feat1.reshape(n, 256, hw1).transpose(0, 2, 1).reshape(n * hw1, 256)
    f2T = feat2.reshape(n, 256, hw2).transpose(0, 2, 1).reshape(n * hw2, 256)
    def pad32(w):
        return jnp.pad(w, ((0, 0), (0, 32 - _C)))

    params = (
        fc1_w[:, :256], fc1_w[:, 256:512], pad32(fc1_w[:, 512:]),
        fc1_b.reshape(1, 256),
        fc2_w[:, :256], pad32(fc2_w[:, 256:]), fc2_b.reshape(1, 256),
        fc3_w[:, :256], pad32(fc3_w[:, 256:]), fc3_b.reshape(1, 256),
        pred_w[:, :256], pad32(pred_w[:, 256:]),
        jnp.broadcast_to(pred_b[:, None], (_C, 128)),
    )
    seg = coarse_logits
    for G in (2, 8):
        hprev = seg.shape[2]
        W2 = 2 * hprev
        HW = W2 * W2
        seg_up, skey = _upsample_unc(seg, G)
        thr = _threshold(skey, _P)
        idxf = _compact(skey, thr, _P)(skey, thr)
        idx = idxf.reshape(n, _P + 256)[:, :_P]
        seg_pm = jnp.pad(seg_up.transpose(0, 2, 3, 1),
                         ((0, 0), (0, 0), (0, 0), (0, 32 - _C))
                         ).reshape(n * HW, 32)
        f2g, f1g, cPg = _gather(n, W2, hw1, h1, W2 // h1, hw2, h2f, W2 // h2f,
                                _P)(f2T, f1T, seg_pm, idx)
        plT = _mlp(f2g, f1g, cPg, params, _P)
        out_flat = _scatter(n, HW, _P)(seg_up.reshape(n * _C * HW),
                                       plT.reshape(n * _C * _P // 128, 128),
                                       idx)
        seg = out_flat.reshape(n, _C, W2, W2)
    return seg


# speedup vs baseline: 16.9452x; 8.9563x over previous
import functools

import numpy as np
import jax
import jax.numpy as jnp
from jax import lax
from jax.experimental import pallas as pl
from jax.experimental.pallas import tpu as pltpu
from jax.experimental.pallas import tpu_sc as plsc

_C = 19
_P = 8192
_IMIN = np.int32(-2147483648)


def _up_unc_body(G, seg_ref, cm_ref, up_ref, skey_ref):
    j = pl.program_id(1)
    C, h, w = seg_ref.shape[1], seg_ref.shape[2], seg_ref.shape[3]
    hs = up_ref.shape[2]
    W2 = 2 * w
    i0 = j * hs
    A = seg_ref[0, :, pl.ds(i0, hs), :]
    up_start = pl.multiple_of(jnp.maximum(i0 - 8, 0), 8)
    Wu = seg_ref[0, :, pl.ds(up_start, hs + 8), :]
    S0 = jnp.concatenate([A[:, :1, :], A[:, : hs - 1, :]], axis=1)
    Am1 = jnp.where(j == 0, S0, Wu[:, 7 : hs + 7, :])
    dn_start = pl.multiple_of(jnp.minimum(i0, h - hs - 8), 8)
    Wd = seg_ref[0, :, pl.ds(dn_start, hs + 8), :]
    S1 = jnp.concatenate([A[:, 1:, :], A[:, hs - 1 :, :]], axis=1)
    Ap1 = jnp.where(j == G - 1, S1, Wd[:, 1 : hs + 1, :])
    E = 0.25 * Am1 + 0.75 * A
    O = 0.75 * A + 0.25 * Ap1
    M = cm_ref[...]
    outs = []
    for par, V in ((0, E), (1, O)):
        V2 = V.reshape(C * hs, w)
        R = lax.dot_general(V2, M, (((1,), (0,)), ((), ())),
                            precision=lax.Precision.HIGHEST,
                            preferred_element_type=jnp.float32)
        R = R.reshape(C, hs, W2)
        up_ref[0, :, :, par, :] = R
        outs.append(R)
    for par, R in ((0, outs[0]), (1, outs[1])):
        m1 = jnp.maximum(R[0], R[1])
        m2 = jnp.minimum(R[0], R[1])
        for c in range(2, C):
            v = R[c]
            m2 = jnp.maximum(m2, jnp.minimum(m1, v))
            m1 = jnp.maximum(m1, v)
        unc = m2 - m1
        ib = lax.bitcast_convert_type(unc, jnp.int32)
        skey = jnp.where(ib >= 0, ib, jnp.bitwise_xor(~ib, jnp.int32(_IMIN)))
        skey_ref[0, :, par, :] = skey


def _col_matrix(w):
    W2 = 2 * w
    t = np.arange(W2)
    x0 = (t - 1) >> 1
    x1 = np.minimum(x0 + 1, w - 1)
    x0c = np.maximum(x0, 0)
    wx = np.where(t % 2 == 0, 0.75, 0.25).astype(np.float32)
    M = np.zeros((w, W2), np.float32)
    M[x0c, t] += 1.0 - wx
    M[x1, t] += wx
    return jnp.asarray(M)


def _upsample_unc(seg, G):
    n, C, h, w = seg.shape
    W2 = 2 * w
    hs = h // G
    M = _col_matrix(w)
    up5, skey4 = pl.pallas_call(
        functools.partial(_up_unc_body, G),
        grid=(n, G),
        in_specs=[
            pl.BlockSpec((1, C, h, w), lambda b, j: (b, 0, 0, 0)),
            pl.BlockSpec((w, W2), lambda b, j: (0, 0)),
        ],
        out_specs=[
            pl.BlockSpec((1, C, hs, 2, W2), lambda b, j: (b, 0, j, 0, 0)),
            pl.BlockSpec((1, hs, 2, W2), lambda b, j: (b, j, 0, 0)),
        ],
        out_shape=[
            jax.ShapeDtypeStruct((n, C, h, 2, W2), jnp.float32),
            jax.ShapeDtypeStruct((n, h, 2, W2), jnp.int32),
        ],
    )(seg, M)
    return up5.reshape(n, C, 2 * h, W2), skey4.reshape(n, 2 * h * W2)


def _thr_body(k, skey_ref, thr_ref):
    sk = skey_ref[0]
    imin = jnp.int32(_IMIN)

    def step(it, t_u):
        cand = t_u | (jnp.int32(1) << (31 - it))
        cand_s = cand ^ imin
        cnt = jnp.sum((sk >= cand_s).astype(jnp.int32))
        return jnp.where(cnt >= k, cand, t_u)

    t_u = lax.fori_loop(0, 32, step, jnp.int32(0))
    T = t_u ^ imin
    cnt_gt = jnp.sum((sk > T).astype(jnp.int32))
    lanes = lax.broadcasted_iota(jnp.int32, (1, 128), 1)
    thr_ref[0] = jnp.where(lanes == 0, T, jnp.where(lanes == 1, cnt_gt, 0))


def _threshold(skey, k):
    n, N = skey.shape
    sk3 = skey.reshape(n, N // 128, 128)
    thr = pl.pallas_call(
        functools.partial(_thr_body, k),
        grid=(n,),
        in_specs=[pl.BlockSpec((1, N // 128, 128), lambda b: (b, 0, 0))],
        out_specs=pl.BlockSpec((1, 1, 128), lambda b: (b, 0, 0)),
        out_shape=jax.ShapeDtypeStruct((n, 1, 128), jnp.int32),
    )(sk3)
    return thr.reshape(n, 128)


def _compact(skey, thr, k):
    n, N = skey.shape
    NW = 16
    CH = N // NW
    R = CH // 16
    PAD = k + 128 * NW
    mesh = plsc.VectorSubcoreMesh(core_axis_name="c", subcore_axis_name="s",
                                  num_cores=1)

    @functools.partial(
        pl.kernel, mesh=mesh,
        compiler_params=pltpu.CompilerParams(needs_layout_passes=False,
                                             use_tc_tiling_on_sc=False),
        out_type=jax.ShapeDtypeStruct((n * PAD,), jnp.int32),
        scratch_types=[
            pltpu.VMEM((CH,), jnp.int32),
            pltpu.VMEM((CH + 16,), jnp.int32),
            pltpu.VMEM((CH + 16,), jnp.int32),
            pltpu.VMEM((CH // 128 + 1, 128), jnp.int32),
            pltpu.VMEM((CH // 128 + 1, 128), jnp.int32),
            pltpu.VMEM((1, 128), jnp.int32),
            pltpu.VMEM((128,), jnp.int32),
            pltpu.VMEM((16, 16), jnp.int32),
            pltpu.VMEM((16,), jnp.int32),
            pltpu.VMEM_SHARED((16, 16), jnp.int32),
            pltpu.SemaphoreType.DMA,
        ],
    )
    def kern(skey_hbm, thr_hbm, out_hbm, chunk_v, pk_g, pk_e, d2_g, d2_e,
             pos_v, thr_v, cnts_v, my_v, shared, sem):
        wid = lax.axis_index("s")
        base = wid * CH
        lane = lax.iota(jnp.int32, 16)
        kb = jnp.broadcast_to(jnp.int32(k), (16,))
        for b in range(n):
            pltpu.sync_copy(skey_hbm.at[b, pl.ds(base, CH)], chunk_v)
            pltpu.sync_copy(thr_hbm.at[b], thr_v)
            T = thr_v[pl.ds(0, 16)][0]

            def pass1(i, carry):
                ngt, neq = carry
                v = chunk_v[pl.ds(i * 16, 16)]
                gidx = base + i * 16 + lane
                m_gt = v > T
                m_eq = v == T
                pg = plsc.all_reduce_population_count(m_gt)
                pe = plsc.all_reduce_population_count(m_eq)

                @pl.when(pg[0] > 0)
                def _():
                    _, sv = plsc.sort_key_val((~m_gt).astype(jnp.int32), gidx)
                    pk_g[pl.ds(ngt, 16)] = sv

                @pl.when(pe[0] > 0)
                def _():
                    _, sv = plsc.sort_key_val((~m_eq).astype(jnp.int32), gidx)
                    pk_e[pl.ds(neq, 16)] = sv

                return ngt + pg[0], neq + pe[0]

            ngt, neq = lax.fori_loop(0, R, pass1,
                                     (jnp.int32(0), jnp.int32(0)))
            mine = (jnp.broadcast_to(ngt, (16,))
                    * (lane == 0).astype(jnp.int32)
                    + jnp.broadcast_to(neq, (16,))
                    * (lane == 1).astype(jnp.int32))
            my_v[...] = mine
            pltpu.sync_copy(my_v, shared.at[wid])
            plsc.subcore_barrier()
            pltpu.sync_copy(shared, cnts_v)
            off_gt = jnp.int32(0)
            off_eq = jnp.int32(0)
            tot_gt = jnp.int32(0)
            for wv in range(NW):
                row = cnts_v[wv]
                g = row[0]
                e = row[1]
                off_gt = off_gt + jnp.where(wv < wid, g, 0)
                off_eq = off_eq + jnp.where(wv < wid, e, 0)
                tot_gt = tot_gt + g

            def densify(pk, d2, cnt):
                def drow(r, _):
                    for t in range(8):
                        d2[r, pl.ds(t * 16, 16)] = pk[pl.ds(r * 128 + t * 16,
                                                            16)]
                    return 0

                lax.fori_loop(0, (cnt + 127) // 128, drow, 0)

            densify(pk_g, d2_g, ngt)
            densify(pk_e, d2_e, neq)

            def emit(d2, cnt, off, boff):
                def row(r, _):
                    for t in range(8):
                        loc = r * 128 + t * 16 + lane
                        posv = off + loc
                        ok = (loc < jnp.broadcast_to(cnt, (16,))) \
                            & (posv < kb)
                        pos_v[0, pl.ds(t * 16, 16)] = jnp.where(
                            ok, posv, k + wid * 128 + t * 16 + lane) + boff
                    pltpu.async_copy(d2.at[r], out_hbm.at[pos_v.at[0]],
                                     sem).wait()
                    return 0

                lax.fori_loop(0, (cnt + 127) // 128, row, 0)

            emit(d2_g, ngt, off_gt, b * PAD)
            emit(d2_e, neq, tot_gt + off_eq, b * PAD)
            plsc.subcore_barrier()

    return kern


def _gather(n, W2, hw1, w1, r1, hw2, w2, r2, k):
    NW = 32
    PPW = k // NW
    mesh = plsc.VectorSubcoreMesh(core_axis_name="c", subcore_axis_name="s")
    HW = W2 * W2
    lw2 = int(np.log2(W2))
    CP = 32

    @functools.partial(
        pl.kernel, mesh=mesh,
        compiler_params=pltpu.CompilerParams(needs_layout_passes=False,
                                             use_tc_tiling_on_sc=False),
        out_type=[
            jax.ShapeDtypeStruct((n, k, 256), jnp.float32),
            jax.ShapeDtypeStruct((n, k, 256), jnp.float32),
            jax.ShapeDtypeStruct((n, k, CP), jnp.float32),
        ],
        scratch_types=[
            pltpu.VMEM((PPW,), jnp.int32),
            pltpu.VMEM((16, 256), jnp.float32),
            pltpu.VMEM((16, 256), jnp.float32),
            pltpu.VMEM((16, 256), jnp.float32),
            pltpu.VMEM((16, 256), jnp.float32),
            pltpu.VMEM((16, 256), jnp.float32),
            pltpu.VMEM((16, CP), jnp.float32),
            pltpu.SemaphoreType.DMA,
        ],
    )
    def kern(f2_hbm, f1_hbm, seg_hbm, idx_hbm, o2_hbm, o1_hbm, oc_hbm,
             idx_v, r00, r01, r10, r11, out_v, cg_v, sem):
        core = lax.axis_index("c")
        sub = lax.axis_index("s")
        wid = sub * 2 + core
        base = wid * PPW
        lane = lax.iota(jnp.int32, 16)
        for b in range(n):
            pltpu.sync_copy(idx_hbm.at[b, pl.ds(base, PPW)], idx_v)
            for f_hbm, o_hbm, hwf, wf, r in ((f2_hbm, o2_hbm, hw2, w2, r2),
                                             (f1_hbm, o1_hbm, hw1, w1, r1)):
                lr = int(np.log2(2 * r))
                hf = hwf // wf
                inv = np.float32(1.0 / (2 * r))

                def chunk(ji, _):
                    iv = idx_v[pl.ds(ji * 16, 16)]
                    ix = iv & (W2 - 1)
                    iy = lax.shift_right_arithmetic(iv, lw2)
                    tx = 2 * ix + (1 - r)
                    x0 = lax.shift_right_arithmetic(tx, lr)
                    fx = (tx - lax.shift_left(x0, lr)).astype(jnp.float32) * inv
                    ty = 2 * iy + (1 - r)
                    y0 = lax.shift_right_arithmetic(ty, lr)
                    fy = (ty - lax.shift_left(y0, lr)).astype(jnp.float32) * inv
                    x0c = jnp.maximum(x0, 0)
                    x1c = jnp.minimum(x0 + 1, wf - 1)
                    y0c = jnp.maximum(y0, 0)
                    y1c = jnp.minimum(y0 + 1, hf - 1)
                    vx0 = (x0 >= 0).astype(jnp.float32)
                    vx1 = (x0 + 1 <= wf - 1).astype(jnp.float32)
                    vy0 = (y0 >= 0).astype(jnp.float32)
                    vy1 = (y0 + 1 <= hf - 1).astype(jnp.float32)
                    wx0 = (1.0 - fx) * vx0
                    wx1 = fx * vx1
                    wy0 = (1.0 - fy) * vy0
                    wy1 = fy * vy1
                    w00 = wx0 * wy0
                    w01 = wx1 * wy0
                    w10 = wx0 * wy1
                    w11 = wx1 * wy1
                    rb = b * hwf + y0c * wf
                    rt = b * hwf + y1c * wf
                    d00 = pltpu.async_copy(f_hbm.at[rb + x0c], r00, sem)
                    d01 = pltpu.async_copy(f_hbm.at[rb + x1c], r01, sem)
                    d10 = pltpu.async_copy(f_hbm.at[rt + x0c], r10, sem)
                    d11 = pltpu.async_copy(f_hbm.at[rt + x1c], r11, sem)
                    d00.wait()
                    d01.wait()
                    d10.wait()
                    d11.wait()

                    def comb(cc, _):
                        s = cc * 16
                        for p in range(16):
                            acc = (r00[p, pl.ds(s, 16)] * w00[p]
                                   + r01[p, pl.ds(s, 16)] * w01[p]
                                   + r10[p, pl.ds(s, 16)] * w10[p]
                                   + r11[p, pl.ds(s, 16)] * w11[p])
                            out_v[p, pl.ds(s, 16)] = acc
                        return 0

                    lax.fori_loop(0, 16, comb, 0)
                    pltpu.sync_copy(out_v, o_hbm.at[b, pl.ds(base + ji * 16, 16)])
                    return 0

                lax.fori_loop(0, PPW // 16, chunk, 0)
            def cchunk(ji, _):
                iv = idx_v[pl.ds(ji * 16, 16)] + b * HW
                pltpu.async_copy(seg_hbm.at[iv], cg_v, sem).wait()
                pltpu.sync_copy(cg_v, oc_hbm.at[b, pl.ds(base + ji * 16, 16)])
                return 0

            lax.fori_loop(0, PPW // 16, cchunk, 0)

    return kern


def _scatter(n, HW, k):
    NW = 16
    PPW = k // NW
    TOT = n * _C * HW
    span = TOT // NW
    CS = HW // 8
    NQ = PPW // 128
    PR = k // 128
    mesh = plsc.VectorSubcoreMesh(core_axis_name="c", subcore_axis_name="s",
                                  num_cores=1)

    @functools.partial(
        pl.kernel, mesh=mesh,
        compiler_params=pltpu.CompilerParams(needs_layout_passes=False,
                                             use_tc_tiling_on_sc=False),
        out_type=jax.ShapeDtypeStruct((TOT,), jnp.float32),
        scratch_types=[
            pltpu.VMEM((CS,), jnp.float32),
            pltpu.VMEM((PPW,), jnp.int32),
            pltpu.VMEM((_C * NQ, 128), jnp.int32),
            pltpu.VMEM((_C * NQ, 128), jnp.float32),
            pltpu.SemaphoreType.DMA,
        ],
    )
    def kern(seg_hbm, pl2_hbm, idx_hbm, out_hbm, buf_v, idx_v, ci_v, val_v,
             sem):
        wid = lax.axis_index("s")

        def cp(i, _):
            start = wid * span + i * CS
            pltpu.sync_copy(seg_hbm.at[pl.ds(start, CS)], buf_v)
            pltpu.sync_copy(buf_v, out_hbm.at[pl.ds(start, CS)])
            return 0

        lax.fori_loop(0, span // CS, cp, 0)
        plsc.subcore_barrier()
        base = wid * PPW
        for b in range(n):
            pltpu.sync_copy(idx_hbm.at[b, pl.ds(base, PPW)], idx_v)
            for c in range(_C):
                row0 = (b * _C + c) * PR + base // 128
                pltpu.sync_copy(pl2_hbm.at[pl.ds(row0, NQ)],
                                val_v.at[pl.ds(c * NQ, NQ)])
                off = (b * _C + c) * HW
                for q in range(NQ):

                    def ci_fill(t, _):
                        ci_v[NQ * c + q, pl.ds(t * 16, 16)] = (
                            idx_v[pl.ds(q * 128 + t * 16, 16)] + off)
                        return 0

                    lax.fori_loop(0, 8, ci_fill, 0)
                dmas = []
                for q in range(NQ):
                    dmas.append(pltpu.async_copy(
                        val_v.at[NQ * c + q],
                        out_hbm.at[ci_v.at[NQ * c + q]], sem))
                for d in dmas:
                    d.wait()

    return kern


def _mlp_body(f2_ref, f1_ref, cp_ref, w1a_ref, w1b_ref, w1c_ref, b1_ref,
              w2h_ref, w2c_ref, b2_ref, w3h_ref, w3c_ref, b3_ref,
              wph_ref, wpc_ref, bp_ref, out_ref):
    x2 = f2_ref[0]
    x1 = f1_ref[0]
    cP = cp_ref[0]
    dnT = (((1,), (1,)), ((), ()))

    def dot(a, b, dn):
        return lax.dot_general(a, b, dn, preferred_element_type=jnp.float32)

    h = dot(x2, w1a_ref[...], dnT) + dot(x1, w1b_ref[...], dnT) \
        + dot(cP, w1c_ref[...], dnT) + b1_ref[...]
    h = jnp.maximum(h, 0.0)
    h = dot(h, w2h_ref[...], dnT) + dot(cP, w2c_ref[...], dnT) + b2_ref[...]
    h = jnp.maximum(h, 0.0)
    h = dot(h, w3h_ref[...], dnT) + dot(cP, w3c_ref[...], dnT) + b3_ref[...]
    h = jnp.maximum(h, 0.0)
    outT = dot(wph_ref[...], h, dnT) \
        + dot(wpc_ref[...], cP, dnT) \
        + bp_ref[...][:, :1]
    out_ref[0] = outT


def _mlp(f2, f1, cP, params, k):
    n = f2.shape[0]
    BLK = 512
    (w1a, w1b, w1c, b1, w2h, w2c, b2, w3h, w3c, b3, wph, wpc, bp) = params
    wspec = lambda shp: pl.BlockSpec(shp, lambda b, j: tuple(0 for _ in shp))
    plT = pl.pallas_call(
        _mlp_body,
        grid=(n, k // BLK),
        in_specs=[
            pl.BlockSpec((1, BLK, 256), lambda b, j: (b, j, 0)),
            pl.BlockSpec((1, BLK, 256), lambda b, j: (b, j, 0)),
            pl.BlockSpec((1, BLK, 32), lambda b, j: (b, j, 0)),
            wspec((256, 256)), wspec((256, 256)), wspec((256, 32)),
            wspec((1, 256)),
            wspec((256, 256)), wspec((256, 32)), wspec((1, 256)),
            wspec((256, 256)), wspec((256, 32)), wspec((1, 256)),
            wspec((_C, 256)), wspec((_C, 32)), wspec((_C, 128)),
        ],
        out_specs=pl.BlockSpec((1, _C, BLK), lambda b, j: (b, 0, j)),
        out_shape=jax.ShapeDtypeStruct((n, _C, k), jnp.float32),
    )(f2, f1, cP, w1a, w1b, w1c, b1, w2h, w2c, b2, w3h, w3c, b3, wph, wpc, bp)
    return plT


def kernel(feat1, feat2, coarse_logits, fc1_w, fc1_b, fc2_w, fc2_b,
           fc3_w, fc3_b, pred_w, pred_b):
    n = feat1.shape[0]
    h1 = feat1.shape[2]
    h2f = feat2.shape[2]
    hw1, hw2 = h1 * h1, h2f * h2f
    f1T = feat1.reshape(n, 256, hw1).transpose(0, 2, 1).reshape(n * hw1, 256)
    f2T = feat2.reshape(n, 256, hw2).transpose(0, 2, 1).reshape(n * hw2, 256)
    def pad32(w):
        return jnp.pad(w, ((0, 0), (0, 32 - _C)))

    params = (
        fc1_w[:, :256], fc1_w[:, 256:512], pad32(fc1_w[:, 512:]),
        fc1_b.reshape(1, 256),
        fc2_w[:, :256], pad32(fc2_w[:, 256:]), fc2_b.reshape(1, 256),
        fc3_w[:, :256], pad32(fc3_w[:, 256:]), fc3_b.reshape(1, 256),
        pred_w[:, :256], pad32(pred_w[:, 256:]),
        jnp.broadcast_to(pred_b[:, None], (_C, 128)),
    )
    seg = coarse_logits
    for G in (2, 8):
        hprev = seg.shape[2]
        W2 = 2 * hprev
        HW = W2 * W2
        seg_up, skey = _upsample_unc(seg, G)
        thr = _threshold(skey, _P)
        idxf = _compact(skey, thr, _P)(skey, thr)
        idx = idxf.reshape(n, _P + 2048)[:, :_P]
        seg_pm = jnp.pad(seg_up.transpose(0, 2, 3, 1),
                         ((0, 0), (0, 0), (0, 0), (0, 32 - _C))
                         ).reshape(n * HW, 32)
        f2g, f1g, cPg = _gather(n, W2, hw1, h1, W2 // h1, hw2, h2f, W2 // h2f,
                                _P)(f2T, f1T, seg_pm, idx)
        plT = _mlp(f2g, f1g, cPg, params, _P)
        out_flat = _scatter(n, HW, _P)(seg_up.reshape(n * _C * HW),
                                       plT.reshape(n * _C * _P // 128, 128),
                                       idx)
        seg = out_flat.reshape(n, _C, W2, W2)
    return seg
